# baseline (speedup 1.0000x reference)
import jax
import jax.numpy as jnp
from jax import lax
from jax.experimental import pallas as pl
from jax.experimental.pallas import tpu as pltpu
from jax.experimental.pallas import tpu_sc as plsc
import functools

EPS = 1e-12
H = 768
NCHUNK = H // 16
R = 64


def _lanesum(v):
    iota = lax.iota(jnp.int32, 16)
    dnums = lax.GatherDimensionNumbers(
        offset_dims=(), collapsed_slice_dims=(0,), start_index_map=(0,))
    for k in (8, 4, 2, 1):
        idx = lax.bitwise_xor(iota, jnp.int32(k))
        v = v + lax.gather(v, idx[:, None], dnums, (1,),
                           mode=lax.GatherScatterMode.PROMISE_IN_BOUNDS)
    return v


def _rsqrt16(v):
    i = lax.bitcast_convert_type(v, jnp.int32)
    y = lax.bitcast_convert_type(
        jnp.int32(0x5F3759DF) - lax.shift_right_arithmetic(i, 1), jnp.float32)
    for _ in range(4):
        y = y * (1.5 - 0.5 * v * y * y)
    return y


def kernel(x, pos_table, gamma, beta):
    b, s, hdim = x.shape
    nrows = b * s
    info = plsc.get_sparse_core_info()
    nw = info.num_cores * info.num_subcores
    rows_pw = nrows // nw
    nsteps = rows_pw // R

    xf = x.reshape(-1)
    posf = pos_table.reshape(-1)

    mesh = plsc.VectorSubcoreMesh(core_axis_name="c", subcore_axis_name="s")

    @functools.partial(
        pl.kernel,
        out_type=jax.ShapeDtypeStruct((nrows * hdim,), jnp.float32),
        mesh=mesh,
        scratch_types=[
            pltpu.VMEM((R * H,), jnp.float32),
            pltpu.VMEM((R * H,), jnp.float32),
            pltpu.VMEM((H,), jnp.float32),
            pltpu.VMEM((H,), jnp.float32),
        ],
    )
    def sc_k(x_hbm, pos_hbm, gamma_hbm, beta_hbm, out_hbm, xbuf, pbuf, gbuf, bbuf):
        wid = lax.axis_index("s") * info.num_cores + lax.axis_index("c")
        base = wid * rows_pw * H
        pbase = (wid * rows_pw % s) * H
        pltpu.sync_copy(gamma_hbm, gbuf)
        pltpu.sync_copy(beta_hbm, bbuf)

        def step(si, _):
            off = base + si * R * H
            poff = pbase + si * R * H
            pltpu.sync_copy(x_hbm.at[pl.ds(off, R * H)], xbuf)
            pltpu.sync_copy(pos_hbm.at[pl.ds(poff, R * H)], pbuf)

            def row(ri, _):
                rb = ri * H

                def chunk1(j, acc):
                    a1, a2 = acc
                    v = xbuf[pl.ds(rb + j * 16, 16)] + pbuf[pl.ds(rb + j * 16, 16)]
                    xbuf[pl.ds(rb + j * 16, 16)] = v
                    return a1 + v, a2 + v * v

                a1, a2 = lax.fori_loop(
                    0, NCHUNK, chunk1,
                    (jnp.zeros((16,), jnp.float32), jnp.zeros((16,), jnp.float32)))
                mean_v = _lanesum(a1) * (1.0 / H)
                var_v = _lanesum(a2) * (1.0 / H) - mean_v * mean_v
                inv_v = _rsqrt16(var_v + EPS)

                def chunk2(j, _):
                    sl = pl.ds(rb + j * 16, 16)
                    gl = pl.ds(j * 16, 16)
                    xbuf[sl] = (xbuf[sl] - mean_v) * inv_v * gbuf[gl] + bbuf[gl]
                    return 0

                lax.fori_loop(0, NCHUNK, chunk2, 0)
                return 0

            lax.fori_loop(0, R, row, 0)
            pltpu.sync_copy(xbuf, out_hbm.at[pl.ds(off, R * H)])
            return 0

        lax.fori_loop(0, nsteps, step, 0)

    out = sc_k(xf, posf, gamma, beta)
    return out.reshape(b, s, hdim)


# speedup vs baseline: 14.1255x; 14.1255x over previous
import jax
import jax.numpy as jnp
from jax.experimental import pallas as pl

EPS = 1e-12
BLK = 512


def _ln_kernel(x_ref, pos_ref, gamma_ref, beta_ref, out_ref):
    h = x_ref.shape[-1]
    pos = pos_ref[...]
    gamma = gamma_ref[...]
    beta = beta_ref[...]
    for bi in range(x_ref.shape[0]):
        e = x_ref[bi] + pos
        mean = jnp.sum(e, axis=-1, keepdims=True) * (1.0 / h)
        d = e - mean
        var = jnp.sum(d * d, axis=-1, keepdims=True) * (1.0 / h)
        inv = jax.lax.rsqrt(var + EPS)
        out_ref[bi] = d * inv * gamma + beta


def kernel(x, pos_table, gamma, beta):
    b, s, hdim = x.shape
    gamma2 = gamma.reshape(1, hdim)
    beta2 = beta.reshape(1, hdim)
    grid = (s // BLK,)
    return pl.pallas_call(
        _ln_kernel,
        grid=grid,
        in_specs=[
            pl.BlockSpec((b, BLK, hdim), lambda i: (0, i, 0)),
            pl.BlockSpec((BLK, hdim), lambda i: (i, 0)),
            pl.BlockSpec((1, hdim), lambda i: (0, 0)),
            pl.BlockSpec((1, hdim), lambda i: (0, 0)),
        ],
        out_specs=pl.BlockSpec((b, BLK, hdim), lambda i: (0, i, 0)),
        out_shape=jax.ShapeDtypeStruct((b, s, hdim), x.dtype),
    )(x, pos_table, gamma2, beta2)
